# baseline (speedup 1.0000x reference)
import functools

import jax
import jax.numpy as jnp
from jax import lax
from jax.experimental import pallas as pl
from jax.experimental.pallas import tpu as pltpu
from jax.experimental.pallas import tpu_sc as plsc

_NC = 2
_NS = 16
_NW = _NC * _NS
_NBUF = 4


def _emb_call(B, S, D, s_per_w, idx, weight):
    mesh = plsc.VectorSubcoreMesh(core_axis_name="c", subcore_axis_name="s")
    n_per_w = s_per_w * S

    @functools.partial(
        pl.kernel,
        out_type=jax.ShapeDtypeStruct((B, S, D), jnp.float32),
        mesh=mesh,
        scratch_types=[
            pltpu.VMEM((n_per_w,), jnp.int32),
            [pltpu.VMEM((S, D), jnp.float32) for _ in range(_NBUF)],
            [pltpu.SemaphoreType.DMA for _ in range(_NBUF)],
        ],
        compiler_params=pltpu.CompilerParams(use_tc_tiling_on_sc=False),
    )
    def emb(idx_hbm, table_hbm, out_hbm, idx_v, rows, gsem):
        wid = lax.axis_index("s") * _NC + lax.axis_index("c")
        seq_base = wid * s_per_w

        pltpu.sync_copy(idx_hbm.at[pl.ds(seq_base * S, n_per_w)], idx_v)
        for b in range(_NBUF):
            pltpu.async_copy(
                table_hbm.at[idx_v.at[pl.ds(b * S, S)]], rows[b], gsem[b])

        def outer(jo, carry):
            i0 = jo * _NBUF
            for b in range(_NBUF):
                i = i0 + b
                pltpu.make_async_copy(
                    table_hbm.at[pl.ds(0, S)], rows[b], gsem[b]).wait()
                pltpu.sync_copy(rows[b], out_hbm.at[seq_base + i])

                @pl.when(i + _NBUF < s_per_w)
                def _():
                    nxt = i + _NBUF
                    pltpu.async_copy(
                        table_hbm.at[idx_v.at[pl.ds(nxt * S, S)]],
                        rows[b], gsem[b])
            return carry

        lax.fori_loop(0, s_per_w // _NBUF, outer, 0)

    return emb(idx, weight)


def kernel(input_ids, weight):
    B, S = input_ids.shape
    V, D = weight.shape
    idx = input_ids.reshape(B * S).astype(jnp.int32)

    s_per_w = B // _NW

    return _emb_call(B, S, D, s_per_w, idx, weight)


# speedup vs baseline: 1.1044x; 1.1044x over previous
import functools

import jax
import jax.numpy as jnp
from jax import lax
from jax.experimental import pallas as pl
from jax.experimental.pallas import tpu as pltpu
from jax.experimental.pallas import tpu_sc as plsc

_NC = 2
_NS = 16
_NW = _NC * _NS
_NBUF = 2
_LANES = 16


def _emb_call(B, S, D, DP, s_per_w, idx, weight_p):
    mesh = plsc.VectorSubcoreMesh(core_axis_name="c", subcore_axis_name="s")
    n_per_w = s_per_w * S

    @functools.partial(
        pl.kernel,
        out_type=jax.ShapeDtypeStruct((B, S, D), jnp.float32),
        mesh=mesh,
        scratch_types=[
            pltpu.VMEM((n_per_w,), jnp.int32),
            [pltpu.VMEM((S, DP), jnp.float32) for _ in range(_NBUF)],
            pltpu.VMEM((S, D), jnp.float32),
            [pltpu.SemaphoreType.DMA for _ in range(_NBUF)],
        ],
    )
    def emb(idx_hbm, table_hbm, out_hbm, idx_v, gbuf, sbuf, gsem):
        wid = lax.axis_index("s") * _NC + lax.axis_index("c")
        seq_base = wid * s_per_w

        pltpu.sync_copy(idx_hbm.at[pl.ds(seq_base * S, n_per_w)], idx_v)
        for b in range(_NBUF):
            pltpu.async_copy(
                table_hbm.at[idx_v.at[pl.ds(b * S, S)]], gbuf[b], gsem[b])

        def outer(jo, carry):
            i0 = jo * _NBUF
            for b in range(_NBUF):
                i = i0 + b
                pltpu.make_async_copy(
                    table_hbm.at[pl.ds(0, S)], gbuf[b], gsem[b]).wait()

                def vcopy(r, c):
                    for k in range(D // _LANES):
                        sbuf[r, pl.ds(k * _LANES, _LANES)] = (
                            gbuf[b][r, pl.ds(k * _LANES, _LANES)])
                    return c

                lax.fori_loop(0, S, vcopy, 0)
                pltpu.sync_copy(sbuf, out_hbm.at[seq_base + i])

                @pl.when(i + _NBUF < s_per_w)
                def _():
                    nxt = i + _NBUF
                    pltpu.async_copy(
                        table_hbm.at[idx_v.at[pl.ds(nxt * S, S)]],
                        gbuf[b], gsem[b])
            return carry

        lax.fori_loop(0, s_per_w // _NBUF, outer, 0)

    return emb(idx, weight_p)


def kernel(input_ids, weight):
    B, S = input_ids.shape
    V, D = weight.shape
    DP = 128
    idx = input_ids.reshape(B * S).astype(jnp.int32)
    weight_p = jnp.pad(weight, ((0, 0), (0, DP - D)))

    s_per_w = B // _NW

    return _emb_call(B, S, D, DP, s_per_w, idx, weight_p)


# speedup vs baseline: 1.3143x; 1.1901x over previous
import functools

import jax
import jax.numpy as jnp
from jax import lax
from jax.experimental import pallas as pl
from jax.experimental.pallas import tpu as pltpu
from jax.experimental.pallas import tpu_sc as plsc

_NC = 2
_NS = 16
_NW = _NC * _NS
_NBUF = 2
_LANES = 16


def _emb_call(B, S, D, DP, s_per_w, idx, weight_p):
    mesh = plsc.VectorSubcoreMesh(core_axis_name="c", subcore_axis_name="s")
    n_per_w = s_per_w * S

    @functools.partial(
        pl.kernel,
        out_type=jax.ShapeDtypeStruct((B * S, D), jnp.float32),
        mesh=mesh,
        scratch_types=[
            pltpu.VMEM((n_per_w,), jnp.int32),
            [pltpu.VMEM((S, DP), jnp.float32) for _ in range(_NBUF)],
            pltpu.VMEM((S, D), jnp.float32),
            [pltpu.SemaphoreType.DMA for _ in range(_NBUF)],
        ],
    )
    def emb(idx_hbm, table_hbm, out_hbm, idx_v, gbuf, sbuf, gsem):
        wid = lax.axis_index("s") * _NC + lax.axis_index("c")
        seq_base = wid * s_per_w

        pltpu.sync_copy(idx_hbm.at[pl.ds(seq_base * S, n_per_w)], idx_v)
        for b in range(_NBUF):
            pltpu.async_copy(
                table_hbm.at[idx_v.at[pl.ds(b * S, S)]], gbuf[b], gsem[b])

        def outer(jo, carry):
            i0 = jo * _NBUF
            for b in range(_NBUF):
                i = i0 + b
                pltpu.make_async_copy(
                    table_hbm.at[pl.ds(0, S)], gbuf[b], gsem[b]).wait()

                def vcopy(r, c):
                    for k in range(D // _LANES):
                        sbuf[r, pl.ds(k * _LANES, _LANES)] = (
                            gbuf[b][r, pl.ds(k * _LANES, _LANES)])
                    return c

                lax.fori_loop(0, S, vcopy, 0)
                pltpu.sync_copy(sbuf, out_hbm.at[pl.ds((seq_base + i) * S, S)])

                @pl.when(i + _NBUF < s_per_w)
                def _():
                    nxt = i + _NBUF
                    pltpu.async_copy(
                        table_hbm.at[idx_v.at[pl.ds(nxt * S, S)]],
                        gbuf[b], gsem[b])
            return carry

        lax.fori_loop(0, s_per_w // _NBUF, outer, 0)

    return emb(idx, weight_p)


def kernel(input_ids, weight):
    B, S = input_ids.shape
    V, D = weight.shape
    DP = 128
    idx = input_ids.reshape(B * S).astype(jnp.int32)
    weight_p = jnp.pad(weight, ((0, 0), (0, DP - D)))

    s_per_w = B // _NW

    out = _emb_call(B, S, D, DP, s_per_w, idx, weight_p)
    return out.reshape(B, S, D)
